# Initial kernel scaffold; baseline (speedup 1.0000x reference)
import jax
import jax.numpy as jnp
from jax.experimental import pallas as pl


def kernel(images, labels, nodes, local_error, edges):
    raise NotImplementedError("write your pallas kernel here")


# speedup vs baseline: 38.0366x; 38.0366x over previous
import jax
import jax.numpy as jnp
from jax.experimental import pallas as pl
from jax.experimental.pallas import tpu as pltpu

E_B = 0.05
E_N = 0.006
D_DECAY = 0.995
INPUT_DIM = 150528
BATCH = 64
M = BATCH + 2

_NC = 8
_KC = INPUT_DIM // _NC

_PREC = jax.lax.Precision.HIGHEST


def _gram_kernel(n_ref, x_ref, g_ref):
    n = n_ref[...]
    x = x_ref[...]
    dn = (((1,), (1,)), ((), ()))
    gnn = jax.lax.dot_general(n, n, dn, precision=_PREC,
                              preferred_element_type=jnp.float32)
    gnx = jax.lax.dot_general(n, x, dn, precision=_PREC,
                              preferred_element_type=jnp.float32)
    gxx = jax.lax.dot_general(x, x, dn, precision=_PREC,
                              preferred_element_type=jnp.float32)
    top = jnp.concatenate([gnn, gnx], axis=1)
    bot = jnp.concatenate([gnx.T, gxx], axis=1)
    g = jnp.concatenate([top, bot], axis=0)

    @pl.when(pl.program_id(0) == 0)
    def _():
        g_ref[...] = g

    @pl.when(pl.program_id(0) != 0)
    def _():
        g_ref[...] += g


def _recon_kernel(g_ref, n_ref, x_ref, out_ref, err_ref, c_ref):
    @pl.when(pl.program_id(0) == 0)
    def _():
        g = g_ref[...]
        lane = jax.lax.broadcasted_iota(jnp.int32, (1, M), 1)
        f32 = jnp.float32
        c0 = (lane == 0).astype(f32)
        c1 = (lane == 1).astype(f32)
        cg0 = g[0:1, :]
        cg1 = g[1:2, :]
        err = jnp.zeros((1, M), f32)

        def body(i, carry):
            c0, c1, cg0, cg1, err = carry
            p = i + 2
            onehot = (lane == p).astype(f32)
            gp = g_ref[pl.ds(p, 1), :]
            gpp = jnp.sum(gp * onehot)
            d0 = jnp.sum(cg0 * c0) - 2.0 * jnp.sum(cg0 * onehot) + gpp
            d1 = jnp.sum(cg1 * c1) - 2.0 * jnp.sum(cg1 * onehot) + gpp
            is0 = d0 <= d1
            cb = jnp.where(is0, c0, c1)
            cgb = jnp.where(is0, cg0, cg1)
            cs = jnp.where(is0, c1, c0)
            cgs = jnp.where(is0, cg1, cg0)
            cb_new = (1.0 - E_B) * cb + E_B * onehot
            cgb_new = (1.0 - E_B) * cgb + E_B * gp
            cs_new = cs + E_N * (onehot - cb)
            cgs_new = cgs + E_N * (gp - cgb)
            c0n = jnp.where(is0, cb_new, cs_new)
            c1n = jnp.where(is0, cs_new, cb_new)
            cg0n = jnp.where(is0, cgb_new, cgs_new)
            cg1n = jnp.where(is0, cgs_new, cgb_new)
            db = jnp.where(is0, d0, d1)
            bmask = jnp.where(is0, (lane == 0).astype(f32),
                              (lane == 1).astype(f32))
            err = (err + db * bmask) * D_DECAY
            return c0n, c1n, cg0n, cg1n, err

        c0, c1, cg0, cg1, err = jax.lax.fori_loop(
            0, BATCH, body, (c0, c1, cg0, cg1, err))
        err_ref[...] = err
        c_ref[...] = jnp.concatenate(
            [c0, c1, jnp.zeros((6, M), jnp.float32)], axis=0)

    cm = c_ref[...]
    dn = (((1,), (0,)), ((), ()))
    out8 = (jax.lax.dot_general(cm[:, 0:2], n_ref[...], dn, precision=_PREC,
                                preferred_element_type=jnp.float32)
            + jax.lax.dot_general(cm[:, 2:M], x_ref[...], dn, precision=_PREC,
                                  preferred_element_type=jnp.float32))
    out_ref[...] = out8[0:2, :]


def kernel(images, labels, nodes, local_error, edges):
    del labels
    g = pl.pallas_call(
        _gram_kernel,
        grid=(_NC,),
        in_specs=[
            pl.BlockSpec((2, _KC), lambda j: (0, j)),
            pl.BlockSpec((BATCH, _KC), lambda j: (0, j)),
        ],
        out_specs=pl.BlockSpec((M, M), lambda j: (0, 0)),
        out_shape=jax.ShapeDtypeStruct((M, M), jnp.float32),
    )(nodes, images)

    nodes_out, err_row = pl.pallas_call(
        _recon_kernel,
        grid=(_NC,),
        in_specs=[
            pl.BlockSpec((M, M), lambda j: (0, 0)),
            pl.BlockSpec((2, _KC), lambda j: (0, j)),
            pl.BlockSpec((BATCH, _KC), lambda j: (0, j)),
        ],
        out_specs=[
            pl.BlockSpec((2, _KC), lambda j: (0, j)),
            pl.BlockSpec((1, M), lambda j: (0, 0)),
        ],
        out_shape=[
            jax.ShapeDtypeStruct((2, INPUT_DIM), jnp.float32),
            jax.ShapeDtypeStruct((1, M), jnp.float32),
        ],
        scratch_shapes=[pltpu.VMEM((8, M), jnp.float32)],
    )(g, nodes, images)

    local_error_out = err_row[0, 0:2] + local_error * (D_DECAY ** BATCH)
    return nodes_out, local_error_out, edges


# speedup vs baseline: 94.2747x; 2.4785x over previous
import jax
import jax.numpy as jnp
from jax.experimental import pallas as pl
from jax.experimental.pallas import tpu as pltpu

E_B = 0.05
E_N = 0.006
D_DECAY = 0.995
INPUT_DIM = 150528
BATCH = 64
M = BATCH + 2

_NC = 4
_KC = INPUT_DIM // _NC

def _split_hi_lo(v):
    hi = v.astype(jnp.bfloat16)
    lo = (v - hi.astype(jnp.float32)).astype(jnp.bfloat16)
    return hi, lo


def _gram_kernel(n_ref, x_ref, g_ref):
    v = jnp.concatenate([x_ref[...], n_ref[...]], axis=0)
    hi, lo = _split_hi_lo(v)
    dn = (((1,), (1,)), ((), ()))
    d1 = jax.lax.dot_general(hi, hi, dn,
                             preferred_element_type=jnp.float32)
    d2 = jax.lax.dot_general(hi, lo, dn,
                             preferred_element_type=jnp.float32)
    g = d1 + d2 + d2.T

    @pl.when(pl.program_id(0) == 0)
    def _():
        g_ref[...] = g

    @pl.when(pl.program_id(0) != 0)
    def _():
        g_ref[...] += g


def _recon_kernel(g_ref, n_ref, x_ref, out_ref, err_ref, c_ref):
    @pl.when(pl.program_id(0) == 0)
    def _():
        lane = jax.lax.broadcasted_iota(jnp.int32, (1, M), 1)
        f32 = jnp.float32
        c0 = (lane == BATCH).astype(f32)
        c1 = (lane == BATCH + 1).astype(f32)
        cg0 = g_ref[BATCH:BATCH + 1, :]
        cg1 = g_ref[BATCH + 1:BATCH + 2, :]
        err = jnp.zeros((1, M), f32)

        def body(p, carry):
            c0, c1, cg0, cg1, err = carry
            onehot = (lane == p).astype(f32)
            gp = g_ref[pl.ds(p, 1), :]
            gpp = jnp.sum(gp * onehot)
            d0 = jnp.sum(cg0 * c0) - 2.0 * jnp.sum(cg0 * onehot) + gpp
            d1 = jnp.sum(cg1 * c1) - 2.0 * jnp.sum(cg1 * onehot) + gpp
            is0 = d0 <= d1
            cb = jnp.where(is0, c0, c1)
            cgb = jnp.where(is0, cg0, cg1)
            cs = jnp.where(is0, c1, c0)
            cgs = jnp.where(is0, cg1, cg0)
            cb_new = (1.0 - E_B) * cb + E_B * onehot
            cgb_new = (1.0 - E_B) * cgb + E_B * gp
            cs_new = cs + E_N * (onehot - cb)
            cgs_new = cgs + E_N * (gp - cgb)
            c0n = jnp.where(is0, cb_new, cs_new)
            c1n = jnp.where(is0, cs_new, cb_new)
            cg0n = jnp.where(is0, cgb_new, cgs_new)
            cg1n = jnp.where(is0, cgs_new, cgb_new)
            db = jnp.where(is0, d0, d1)
            bmask = jnp.where(is0, (lane == 0).astype(f32),
                              (lane == 1).astype(f32))
            err = (err + db * bmask) * D_DECAY
            return c0n, c1n, cg0n, cg1n, err

        c0, c1, cg0, cg1, err = jax.lax.fori_loop(
            0, BATCH, body, (c0, c1, cg0, cg1, err))
        err_ref[...] = err
        c_ref[...] = jnp.concatenate(
            [c0, c1, jnp.zeros((6, M), jnp.float32)], axis=0)

    cm = c_ref[...]
    v = jnp.concatenate([x_ref[...], n_ref[...]], axis=0)
    hi_v, lo_v = _split_hi_lo(v)
    hi_c, lo_c = _split_hi_lo(cm)
    dn = (((1,), (0,)), ((), ()))
    out8 = (jax.lax.dot_general(hi_c, hi_v, dn,
                                preferred_element_type=jnp.float32)
            + jax.lax.dot_general(hi_c, lo_v, dn,
                                  preferred_element_type=jnp.float32)
            + jax.lax.dot_general(lo_c, hi_v, dn,
                                  preferred_element_type=jnp.float32))
    out_ref[...] = out8[0:2, :]


def kernel(images, labels, nodes, local_error, edges):
    del labels
    g = pl.pallas_call(
        _gram_kernel,
        grid=(_NC,),
        in_specs=[
            pl.BlockSpec((2, _KC), lambda j: (0, j)),
            pl.BlockSpec((BATCH, _KC), lambda j: (0, j)),
        ],
        out_specs=pl.BlockSpec((M, M), lambda j: (0, 0)),
        out_shape=jax.ShapeDtypeStruct((M, M), jnp.float32),
    )(nodes, images)

    nodes_out, err_row = pl.pallas_call(
        _recon_kernel,
        grid=(_NC,),
        in_specs=[
            pl.BlockSpec((M, M), lambda j: (0, 0)),
            pl.BlockSpec((2, _KC), lambda j: (0, j)),
            pl.BlockSpec((BATCH, _KC), lambda j: (0, j)),
        ],
        out_specs=[
            pl.BlockSpec((2, _KC), lambda j: (0, j)),
            pl.BlockSpec((1, M), lambda j: (0, 0)),
        ],
        out_shape=[
            jax.ShapeDtypeStruct((2, INPUT_DIM), jnp.float32),
            jax.ShapeDtypeStruct((1, M), jnp.float32),
        ],
        scratch_shapes=[pltpu.VMEM((8, M), jnp.float32)],
    )(g, nodes, images)

    local_error_out = err_row[0, 0:2] + local_error * (D_DECAY ** BATCH)
    return nodes_out, local_error_out, edges


# speedup vs baseline: 103.0775x; 1.0934x over previous
import jax
import jax.numpy as jnp
from jax.experimental import pallas as pl
from jax.experimental.pallas import tpu as pltpu

E_B = 0.05
E_N = 0.006
D_DECAY = 0.995
INPUT_DIM = 150528
BATCH = 64
M = BATCH + 2

_NC = 8
_KC = INPUT_DIM // _NC


def _split_hi_lo(v):
    hi = v.astype(jnp.bfloat16)
    lo = (v - hi.astype(jnp.float32)).astype(jnp.bfloat16)
    return hi, lo


_DN_T = (((1,), (1,)), ((), ()))
_DN = (((1,), (0,)), ((), ()))


def _dot(a, b, dn):
    return jax.lax.dot_general(a, b, dn, preferred_element_type=jnp.float32)


def _fused_kernel(n_ref, x_ref, out_ref, err_ref,
                  hi_ref, lo_ref, g_ref, c_ref):
    ph = pl.program_id(0)
    j = pl.program_id(1)

    @pl.when(ph == 0)
    def _():
        hi_x, lo_x = _split_hi_lo(x_ref[...])
        hi_n, lo_n = _split_hi_lo(n_ref[...])
        hi_ref[j] = hi_x
        lo_ref[j] = lo_x
        hi = jnp.concatenate([hi_x, hi_n], axis=0)
        lo = jnp.concatenate([lo_x, lo_n], axis=0)
        d1 = _dot(hi, hi, _DN_T)
        d2 = _dot(hi, lo, _DN_T)
        g = d1 + d2 + d2.T

        @pl.when(j == 0)
        def _():
            g_ref[...] = g

        @pl.when(j != 0)
        def _():
            g_ref[...] += g

    @pl.when((ph == 1) & (j == 0))
    def _():
        lane = jax.lax.broadcasted_iota(jnp.int32, (1, M), 1)
        f32 = jnp.float32
        c0 = (lane == BATCH).astype(f32)
        c1 = (lane == BATCH + 1).astype(f32)
        cg0 = g_ref[BATCH:BATCH + 1, :]
        cg1 = g_ref[BATCH + 1:BATCH + 2, :]
        err = jnp.zeros((1, M), f32)

        def body(p, carry):
            c0, c1, cg0, cg1, err = carry
            onehot = (lane == p).astype(f32)
            gp = g_ref[pl.ds(p, 1), :]
            gpp = jnp.sum(gp * onehot)
            d0 = jnp.sum(cg0 * c0) - 2.0 * jnp.sum(cg0 * onehot) + gpp
            d1 = jnp.sum(cg1 * c1) - 2.0 * jnp.sum(cg1 * onehot) + gpp
            is0 = d0 <= d1
            cb = jnp.where(is0, c0, c1)
            cgb = jnp.where(is0, cg0, cg1)
            cs = jnp.where(is0, c1, c0)
            cgs = jnp.where(is0, cg1, cg0)
            cb_new = (1.0 - E_B) * cb + E_B * onehot
            cgb_new = (1.0 - E_B) * cgb + E_B * gp
            cs_new = cs + E_N * (onehot - cb)
            cgs_new = cgs + E_N * (gp - cgb)
            c0n = jnp.where(is0, cb_new, cs_new)
            c1n = jnp.where(is0, cs_new, cb_new)
            cg0n = jnp.where(is0, cgb_new, cgs_new)
            cg1n = jnp.where(is0, cgs_new, cgb_new)
            db = jnp.where(is0, d0, d1)
            bmask = jnp.where(is0, (lane == 0).astype(f32),
                              (lane == 1).astype(f32))
            err = (err + db * bmask) * D_DECAY
            return c0n, c1n, cg0n, cg1n, err

        c0, c1, cg0, cg1, err = jax.lax.fori_loop(
            0, BATCH, body, (c0, c1, cg0, cg1, err))
        err_ref[...] = err
        c_ref[...] = jnp.concatenate(
            [c0, c1, jnp.zeros((6, M), jnp.float32)], axis=0)

    @pl.when(ph == 1)
    def _():
        cm = c_ref[...]
        hi_c, lo_c = _split_hi_lo(cm)
        hi_x = hi_ref[j]
        lo_x = lo_ref[j]
        hi_n, lo_n = _split_hi_lo(n_ref[...])
        out8 = (_dot(hi_c[:, 0:BATCH], hi_x, _DN)
                + _dot(hi_c[:, 0:BATCH], lo_x, _DN)
                + _dot(lo_c[:, 0:BATCH], hi_x, _DN)
                + _dot(hi_c[:, BATCH:M], hi_n, _DN)
                + _dot(hi_c[:, BATCH:M], lo_n, _DN)
                + _dot(lo_c[:, BATCH:M], hi_n, _DN))
        out_ref[...] = out8[0:2, :]


def kernel(images, labels, nodes, local_error, edges):
    del labels
    nodes_out, err_row = pl.pallas_call(
        _fused_kernel,
        grid=(2, _NC),
        in_specs=[
            pl.BlockSpec((2, _KC), lambda p, j: (0, j)),
            pl.BlockSpec((BATCH, _KC),
                         lambda p, j: (0, j * (1 - p) + (_NC - 1) * p)),
        ],
        out_specs=[
            pl.BlockSpec((2, _KC), lambda p, j: (0, j * p)),
            pl.BlockSpec((1, M), lambda p, j: (0, 0)),
        ],
        out_shape=[
            jax.ShapeDtypeStruct((2, INPUT_DIM), jnp.float32),
            jax.ShapeDtypeStruct((1, M), jnp.float32),
        ],
        scratch_shapes=[
            pltpu.VMEM((_NC, BATCH, _KC), jnp.bfloat16),
            pltpu.VMEM((_NC, BATCH, _KC), jnp.bfloat16),
            pltpu.VMEM((M, M), jnp.float32),
            pltpu.VMEM((8, M), jnp.float32),
        ],
    )(nodes, images)

    local_error_out = err_row[0, 0:2] + local_error * (D_DECAY ** BATCH)
    return nodes_out, local_error_out, edges


# speedup vs baseline: 114.9734x; 1.1154x over previous
import jax
import jax.numpy as jnp
from jax.experimental import pallas as pl
from jax.experimental.pallas import tpu as pltpu

E_B = 0.05
E_N = 0.006
D_DECAY = 0.995
INPUT_DIM = 150528
BATCH = 64
M = BATCH + 2

_NC = 4
_KC = INPUT_DIM // _NC


_DN_T = (((1,), (1,)), ((), ()))
_DN = (((1,), (0,)), ((), ()))


def _dot(a, b, dn):
    return jax.lax.dot_general(a, b, dn, preferred_element_type=jnp.float32)


def _fused_kernel(n_ref, x_ref, out_ref, err_ref,
                  hi_ref, g_ref, c_ref):
    ph = pl.program_id(0)
    j = pl.program_id(1)

    @pl.when(ph == 0)
    def _():
        x = x_ref[...]
        n = n_ref[...]
        hi_x = x.astype(jnp.bfloat16)
        hi_n = n.astype(jnp.bfloat16)
        hi_ref[j] = hi_x
        hix32 = hi_x.astype(jnp.float32)
        hin32 = hi_n.astype(jnp.float32)
        hi = jnp.concatenate([hi_x, hi_n], axis=0)
        hi32 = jnp.concatenate([hix32, hin32], axis=0)
        lo = jnp.concatenate([x - hix32, n - hin32], axis=0)
        d1 = _dot(hi, hi, _DN_T)
        d2 = _dot(hi32, lo, _DN_T)
        g = d1 + d2 + d2.T

        @pl.when(j == 0)
        def _():
            g_ref[...] = g

        @pl.when(j != 0)
        def _():
            g_ref[...] += g

    @pl.when((ph == 1) & (j == 0))
    def _():
        lane = jax.lax.broadcasted_iota(jnp.int32, (1, M), 1)
        f32 = jnp.float32
        c0 = (lane == BATCH).astype(f32)
        c1 = (lane == BATCH + 1).astype(f32)
        cg0 = g_ref[BATCH:BATCH + 1, :]
        cg1 = g_ref[BATCH + 1:BATCH + 2, :]
        err = jnp.zeros((1, M), f32)

        def body(p, carry):
            c0, c1, cg0, cg1, err = carry
            onehot = (lane == p).astype(f32)
            gp = g_ref[pl.ds(p, 1), :]
            gpp = jnp.sum(gp * onehot)
            d0 = jnp.sum(cg0 * c0) - 2.0 * jnp.sum(cg0 * onehot) + gpp
            d1 = jnp.sum(cg1 * c1) - 2.0 * jnp.sum(cg1 * onehot) + gpp
            is0 = d0 <= d1
            cb = jnp.where(is0, c0, c1)
            cgb = jnp.where(is0, cg0, cg1)
            cs = jnp.where(is0, c1, c0)
            cgs = jnp.where(is0, cg1, cg0)
            cb_new = (1.0 - E_B) * cb + E_B * onehot
            cgb_new = (1.0 - E_B) * cgb + E_B * gp
            cs_new = cs + E_N * (onehot - cb)
            cgs_new = cgs + E_N * (gp - cgb)
            c0n = jnp.where(is0, cb_new, cs_new)
            c1n = jnp.where(is0, cs_new, cb_new)
            cg0n = jnp.where(is0, cgb_new, cgs_new)
            cg1n = jnp.where(is0, cgs_new, cgb_new)
            db = jnp.where(is0, d0, d1)
            bmask = jnp.where(is0, (lane == 0).astype(f32),
                              (lane == 1).astype(f32))
            err = (err + db * bmask) * D_DECAY
            return c0n, c1n, cg0n, cg1n, err

        c0, c1, cg0, cg1, err = jax.lax.fori_loop(
            0, BATCH, body, (c0, c1, cg0, cg1, err))
        err_ref[...] = err
        c_ref[...] = jnp.concatenate(
            [c0, c1, jnp.zeros((6, M), jnp.float32)], axis=0)

    @pl.when(ph == 1)
    def _():
        cm = c_ref[...]
        hi_c = cm.astype(jnp.bfloat16)
        lo_c = (cm - hi_c.astype(jnp.float32)).astype(jnp.bfloat16)
        hi_x = hi_ref[j]
        n = n_ref[...]
        hi_n = n.astype(jnp.bfloat16)
        lo_n = (n - hi_n.astype(jnp.float32)).astype(jnp.bfloat16)
        out8 = (_dot(hi_c[:, 0:BATCH], hi_x, _DN)
                + _dot(lo_c[:, 0:BATCH], hi_x, _DN)
                + _dot(hi_c[:, BATCH:M], hi_n, _DN)
                + _dot(hi_c[:, BATCH:M], lo_n, _DN)
                + _dot(lo_c[:, BATCH:M], hi_n, _DN))
        out_ref[...] = out8[0:2, :]


def kernel(images, labels, nodes, local_error, edges):
    del labels
    nodes_out, err_row = pl.pallas_call(
        _fused_kernel,
        grid=(2, _NC),
        in_specs=[
            pl.BlockSpec((2, _KC), lambda p, j: (0, j)),
            pl.BlockSpec((BATCH, _KC),
                         lambda p, j: (0, j * (1 - p) + (_NC - 1) * p)),
        ],
        out_specs=[
            pl.BlockSpec((2, _KC), lambda p, j: (0, j * p)),
            pl.BlockSpec((1, M), lambda p, j: (0, 0)),
        ],
        out_shape=[
            jax.ShapeDtypeStruct((2, INPUT_DIM), jnp.float32),
            jax.ShapeDtypeStruct((1, M), jnp.float32),
        ],
        scratch_shapes=[
            pltpu.VMEM((_NC, BATCH, _KC), jnp.bfloat16),
            pltpu.VMEM((M, M), jnp.float32),
            pltpu.VMEM((8, M), jnp.float32),
        ],
    )(nodes, images)

    local_error_out = err_row[0, 0:2] + local_error * (D_DECAY ** BATCH)
    return nodes_out, local_error_out, edges
